# Initial kernel scaffold; baseline (speedup 1.0000x reference)
import jax
import jax.numpy as jnp
from jax.experimental import pallas as pl


def kernel(x, Gi, W, b):
    raise NotImplementedError("write your pallas kernel here")


# speedup vs baseline: 2.7902x; 2.7902x over previous
import functools

import jax
import jax.numpy as jnp
from jax import lax
from jax.experimental import pallas as pl
from jax.experimental.pallas import tpu as pltpu
from jax.experimental.pallas import tpu_sc as plsc

_NC, _NS = 2, 16
_NW = _NC * _NS
_CH = 128
_LANES = 16


def _proj_body(x_ref, w0_ref, w1_ref, b_ref, y0_ref, y1_ref):
    xb = x_ref[...]
    dn = (((0,), (1,)), ((), ()))
    y0_ref[...] = lax.dot_general(
        xb, w0_ref[...], dn, preferred_element_type=jnp.float32) + b_ref[...]
    y1_ref[...] = lax.dot_general(
        xb, w1_ref[...], dn, preferred_element_type=jnp.float32)


def _make_sc_kernel(vp, c):
    nchunks = vp // (_NW * _CH)
    mesh = plsc.VectorSubcoreMesh(
        core_axis_name="c", subcore_axis_name="s",
        num_cores=_NC, num_subcores=_NS)

    @functools.partial(
        pl.kernel,
        out_type=jax.ShapeDtypeStruct((vp, c), jnp.float32),
        mesh=mesh,
        scratch_types=[
            pltpu.VMEM((4, _CH), jnp.int32),
            pltpu.VMEM((_CH, c), jnp.float32),
            pltpu.VMEM((_CH, c), jnp.float32),
            pltpu.VMEM((_CH, c), jnp.float32),
            pltpu.VMEM((_CH, c), jnp.float32),
            pltpu.SemaphoreType.DMA,
        ],
    )
    def sc_fn(y0_hbm, y1_hbm, idx_hbm, out_hbm, idxv, ba, bb, bc, bd, sem):
        wid = lax.axis_index("s") * _NC + lax.axis_index("c")
        base_chunk = wid * nchunks

        def chunk(j, carry):
            cidx = base_chunk + j
            pltpu.sync_copy(idx_hbm.at[cidx], idxv)
            cp0 = pltpu.async_copy(y0_hbm.at[idxv.at[0]], ba, sem)
            cp1 = pltpu.async_copy(y1_hbm.at[idxv.at[1]], bb, sem)
            cp2 = pltpu.async_copy(y1_hbm.at[idxv.at[2]], bc, sem)
            cp3 = pltpu.async_copy(y1_hbm.at[idxv.at[3]], bd, sem)
            cp0.wait()
            cp1.wait()
            cp2.wait()
            cp3.wait()

            def row(r, rcarry):
                for q in range(c // _LANES):
                    s = pl.ds(q * _LANES, _LANES)
                    ba[r, s] = ba[r, s] + bb[r, s] + bc[r, s] + bd[r, s]
                return rcarry

            lax.fori_loop(0, _CH, row, 0)
            pltpu.sync_copy(ba, out_hbm.at[pl.ds(cidx * _CH, _CH)])
            return carry

        lax.fori_loop(0, nchunks, chunk, 0)

    return sc_fn


def kernel(x, Gi, W, b):
    bsz, cin, v, _ = x.shape
    cout = W.shape[0]
    x2d = x[0, :, :, 0]
    w0 = W[:, :, 0, 0]
    w1 = W[:, :, 0, 1]
    b2 = b.reshape(1, cout)

    vt = 2048
    y0, y1 = pl.pallas_call(
        _proj_body,
        grid=(pl.cdiv(v, vt),),
        in_specs=[
            pl.BlockSpec((cin, vt), lambda i: (0, i)),
            pl.BlockSpec((cout, cin), lambda i: (0, 0)),
            pl.BlockSpec((cout, cin), lambda i: (0, 0)),
            pl.BlockSpec((1, cout), lambda i: (0, 0)),
        ],
        out_specs=[
            pl.BlockSpec((vt, cout), lambda i: (i, 0)),
            pl.BlockSpec((vt, cout), lambda i: (i, 0)),
        ],
        out_shape=[
            jax.ShapeDtypeStruct((v, cout), jnp.float32),
            jax.ShapeDtypeStruct((v, cout), jnp.float32),
        ],
        compiler_params=pltpu.CompilerParams(
            dimension_semantics=("arbitrary",)),
    )(x2d, w0, w1, b2)

    grain = _NW * _CH
    vp = ((v + grain - 1) // grain) * grain
    idx = Gi[0, :, :4].astype(jnp.int32)
    idx = jnp.pad(idx, ((0, vp - v), (0, 0)))
    idxb = idx.T.reshape(4, vp // _CH, _CH).transpose(1, 0, 2)

    out_t = _make_sc_kernel(vp, cout)(y0, y1, idxb)
    out = out_t[:v].T
    return out[None, :, :, None]
